# baseline (speedup 1.0000x reference)
import functools

import jax
import jax.numpy as jnp
import numpy as np
from jax import lax
from jax.experimental import pallas as pl
from jax.experimental.pallas import tpu as pltpu
from jax.experimental.pallas import tpu_sc as plsc

B, BLOCKS, H, K2 = 32, 4096, 8, 49
SIZE = 7
POS = 2 * SIZE - 1
N = BLOCKS * H * K2


def _pad_bias_np(indexes, total, dilation_rate):
    size = indexes.shape[0]
    left = np.repeat(indexes[: size // 2], dilation_rate)
    right = np.repeat(indexes[size // 2 + 1:], dilation_rate)
    center = np.repeat(indexes[size // 2], total - left.shape[0] - right.shape[0])
    return np.concatenate([left, center, right], axis=-1)


@functools.lru_cache(maxsize=1)
def _flat_gather_indices():
    height = int(np.sqrt(float(BLOCKS)))
    width = BLOCKS // height
    idx_hh = np.arange(0, SIZE)
    idx_ww = np.arange(0, SIZE)
    coords = np.reshape(np.expand_dims(idx_hh, -1) * POS + idx_ww, [-1]).astype(np.int64)
    bias_hh = _pad_bias_np(idx_hh, total=height, dilation_rate=1)
    bias_ww = _pad_bias_np(idx_ww, total=width, dilation_rate=1)
    bias_hw = np.expand_dims(bias_hh, -1) * POS + bias_ww
    bc = np.expand_dims(bias_hw, -1) + coords
    bc = np.reshape(bc, [-1, SIZE ** 2])[::-1]
    flat = (np.arange(H)[None, :, None] * (POS * POS) + bc[:, None, :])
    return np.ascontiguousarray(flat.reshape(-1).astype(np.int32))


def _build_bias_sc(pos_bias_flat, idx_flat):
    info = plsc.get_sparse_core_info()
    nc, ns, lanes = info.num_cores, info.num_subcores, info.num_lanes
    nw = nc * ns
    per_w = N // nw
    assert N % nw == 0 and per_w % lanes == 0 and per_w % 8 == 0
    table_words = POS * POS * H

    mesh = plsc.VectorSubcoreMesh(core_axis_name="c", subcore_axis_name="s")

    @functools.partial(
        pl.kernel,
        mesh=mesh,
        compiler_params=pltpu.CompilerParams(needs_layout_passes=False),
        out_type=jax.ShapeDtypeStruct((N,), jnp.float32),
        scratch_types=[
            pltpu.VMEM((table_words,), jnp.float32),
            pltpu.VMEM((per_w,), jnp.int32),
            pltpu.VMEM((per_w,), jnp.float32),
        ],
    )
    def gather_kernel(table_hbm, idx_hbm, out_hbm, tab_v, idx_v, val_v):
        wid = lax.axis_index("s") * nc + lax.axis_index("c")
        base = wid * per_w
        pltpu.sync_copy(table_hbm, tab_v)
        pltpu.sync_copy(idx_hbm.at[pl.ds(base, per_w)], idx_v)

        def body(i, carry):
            off = i * lanes
            idx = idx_v[pl.ds(off, lanes)]
            val_v[pl.ds(off, lanes)] = plsc.load_gather(tab_v, [idx])
            return carry

        lax.fori_loop(0, per_w // lanes, body, 0, unroll=8)
        pltpu.sync_copy(val_v, out_hbm.at[pl.ds(base, per_w)])

    return gather_kernel(pos_bias_flat, idx_flat)


def _add_bias_tc(inputs, bias3d, nb=256):
    grid = (BLOCKS // nb, B)

    def add_body(x_ref, b_ref, o_ref):
        o_ref[0] = x_ref[0] + b_ref[...]

    return pl.pallas_call(
        add_body,
        grid=grid,
        in_specs=[
            pl.BlockSpec((1, nb, H, K2), lambda j, b: (b, j, 0, 0)),
            pl.BlockSpec((nb, H, K2), lambda j, b: (j, 0, 0)),
        ],
        out_specs=pl.BlockSpec((1, nb, H, K2), lambda j, b: (b, j, 0, 0)),
        out_shape=jax.ShapeDtypeStruct((B, BLOCKS, H, K2), jnp.float32),
    )(inputs, bias3d)


def kernel(inputs, pos_bias):
    idx_flat = jnp.asarray(_flat_gather_indices())
    bias_flat = _build_bias_sc(jnp.reshape(pos_bias, (-1,)), idx_flat)
    bias3d = jnp.reshape(bias_flat, (BLOCKS, H, K2))
    return _add_bias_tc(inputs, bias3d)


# speedup vs baseline: 4.4560x; 4.4560x over previous
import functools

import jax
import jax.numpy as jnp
import numpy as np
from jax import lax
from jax.experimental import pallas as pl
from jax.experimental.pallas import tpu as pltpu
from jax.experimental.pallas import tpu_sc as plsc

B, BLOCKS, H, K2 = 32, 4096, 8, 49
SIZE = 7
POS = 2 * SIZE - 1
N = BLOCKS * H * K2


def _pad_bias_np(indexes, total, dilation_rate):
    size = indexes.shape[0]
    left = np.repeat(indexes[: size // 2], dilation_rate)
    right = np.repeat(indexes[size // 2 + 1:], dilation_rate)
    center = np.repeat(indexes[size // 2], total - left.shape[0] - right.shape[0])
    return np.concatenate([left, center, right], axis=-1)


@functools.lru_cache(maxsize=1)
def _flat_gather_indices():
    height = int(np.sqrt(float(BLOCKS)))
    width = BLOCKS // height
    idx_hh = np.arange(0, SIZE)
    idx_ww = np.arange(0, SIZE)
    coords = np.reshape(np.expand_dims(idx_hh, -1) * POS + idx_ww, [-1]).astype(np.int64)
    bias_hh = _pad_bias_np(idx_hh, total=height, dilation_rate=1)
    bias_ww = _pad_bias_np(idx_ww, total=width, dilation_rate=1)
    bias_hw = np.expand_dims(bias_hh, -1) * POS + bias_ww
    bc = np.expand_dims(bias_hw, -1) + coords
    bc = np.reshape(bc, [-1, SIZE ** 2])[::-1]
    flat = (np.arange(H)[None, :, None] * (POS * POS) + bc.T[:, None, :])
    return np.ascontiguousarray(flat.reshape(-1).astype(np.int32))


def _build_bias_sc(pos_bias_flat, idx_flat):
    info = plsc.get_sparse_core_info()
    nc, ns, lanes = info.num_cores, info.num_subcores, info.num_lanes
    nw = nc * ns
    per_w = N // nw
    assert N % nw == 0 and per_w % lanes == 0 and per_w % 8 == 0
    table_words = POS * POS * H

    mesh = plsc.VectorSubcoreMesh(core_axis_name="c", subcore_axis_name="s")

    @functools.partial(
        pl.kernel,
        mesh=mesh,
        compiler_params=pltpu.CompilerParams(needs_layout_passes=False),
        out_type=jax.ShapeDtypeStruct((N,), jnp.float32),
        scratch_types=[
            pltpu.VMEM((table_words,), jnp.float32),
            pltpu.VMEM((per_w,), jnp.int32),
            pltpu.VMEM((per_w,), jnp.float32),
        ],
    )
    def gather_kernel(table_hbm, idx_hbm, out_hbm, tab_v, idx_v, val_v):
        wid = lax.axis_index("s") * nc + lax.axis_index("c")
        base = wid * per_w
        pltpu.sync_copy(table_hbm, tab_v)
        pltpu.sync_copy(idx_hbm.at[pl.ds(base, per_w)], idx_v)

        def body(i, carry):
            off = i * lanes
            idx = idx_v[pl.ds(off, lanes)]
            val_v[pl.ds(off, lanes)] = plsc.load_gather(tab_v, [idx])
            return carry

        lax.fori_loop(0, per_w // lanes, body, 0, unroll=8)
        pltpu.sync_copy(val_v, out_hbm.at[pl.ds(base, per_w)])

    return gather_kernel(pos_bias_flat, idx_flat)


def _add_bias_tc(x_t, bias_t, c=512):
    grid = (BLOCKS // c, B)

    def add_body(x_ref, b_ref, o_ref):
        o_ref[0] = x_ref[0] + b_ref[...]

    return pl.pallas_call(
        add_body,
        grid=grid,
        in_specs=[
            pl.BlockSpec((1, K2, H, c), lambda j, b: (b, 0, 0, j)),
            pl.BlockSpec((K2, H, c), lambda j, b: (0, 0, j)),
        ],
        out_specs=pl.BlockSpec((1, K2, H, c), lambda j, b: (b, 0, 0, j)),
        out_shape=jax.ShapeDtypeStruct((B, K2, H, BLOCKS), jnp.float32),
    )(x_t, bias_t)


def kernel(inputs, pos_bias):
    idx_flat = jnp.asarray(_flat_gather_indices())
    bias_flat = _build_bias_sc(jnp.reshape(pos_bias, (-1,)), idx_flat)
    bias_t = jnp.reshape(bias_flat, (K2, H, BLOCKS))
    x_t = jnp.transpose(inputs, (0, 3, 2, 1))
    out_t = _add_bias_tc(x_t, bias_t)
    return jnp.transpose(out_t, (0, 3, 2, 1))
